# baseline (speedup 1.0000x reference)
import functools

import jax
import jax.numpy as jnp
from jax.experimental import pallas as pl

_BN = 512


def _cheb_body(nk, relu, t_ref, w_ref, b_ref, o_ref):
    acc = jnp.dot(t_ref[0], w_ref[0], preferred_element_type=jnp.float32)
    for k in range(1, nk):
        acc = acc + jnp.dot(t_ref[k], w_ref[k],
                            preferred_element_type=jnp.float32)
    acc = acc + b_ref[0]
    if relu:
        acc = jnp.maximum(acc, 0.0)
    o_ref[...] = acc


def _cheb_matmul(t_stack, W, b, relu):
    nk, npad, cin = t_stack.shape
    cout = W.shape[2]
    grid = (npad // _BN,)
    body = functools.partial(_cheb_body, nk, relu)
    return pl.pallas_call(
        body,
        grid=grid,
        in_specs=[
            pl.BlockSpec((nk, _BN, cin), lambda i: (0, i, 0)),
            pl.BlockSpec((nk, cin, cout), lambda i: (0, 0, 0)),
            pl.BlockSpec((1, cout), lambda i: (0, 0)),
        ],
        out_specs=pl.BlockSpec((_BN, cout), lambda i: (i, 0)),
        out_shape=jax.ShapeDtypeStruct((npad, cout), jnp.float32),
    )(t_stack, W, b[None, :])


def kernel(x, edge_index, params1, params2, params3):
    n = x.shape[0] // 3
    npad = ((n + _BN - 1) // _BN) * _BN
    src = edge_index[0]
    dst = edge_index[1]

    mask = (src != dst).astype(jnp.float32)
    deg = jax.ops.segment_sum(mask, src, num_segments=n)
    deg_inv_sqrt = jnp.where(deg > 0, 1.0 / jnp.sqrt(jnp.maximum(deg, 1.0)),
                             0.0)
    norm = (-deg_inv_sqrt[src] * deg_inv_sqrt[dst] * mask)[:, None]

    def prop(t):
        return jax.ops.segment_sum(norm * t[src], dst, num_segments=n)

    def branch(h, params):
        nlayers = len(params)
        for i, (W, b) in enumerate(params):
            nk = W.shape[0]
            tx = [h]
            for k in range(1, nk):
                if k == 1:
                    tx.append(prop(h))
                else:
                    tx.append(2.0 * prop(tx[-1]) - tx[-2])
            t_stack = jnp.stack(tx, axis=0)
            t_stack = jnp.pad(t_stack, ((0, 0), (0, npad - n), (0, 0)))
            out = _cheb_matmul(t_stack, W, b, relu=(i < nlayers - 1))
            h = out[:n]
        return h

    x1 = branch(x[0::3], params1)
    x2 = branch(x[1::3], params2)
    x3 = branch(x[2::3], params3)
    return jnp.stack([x1, x2, x3], axis=1).reshape(3 * n, x1.shape[1])


# speedup vs baseline: 1.9226x; 1.9226x over previous
import functools

import jax
import jax.numpy as jnp
from jax.experimental import pallas as pl

_BN = 256


def _cheb_body(nk, relu, t_ref, w_ref, b_ref, o_ref):
    acc = jnp.dot(t_ref[0], w_ref[0], preferred_element_type=jnp.float32)
    for k in range(1, nk):
        acc = acc + jnp.dot(t_ref[k], w_ref[k],
                            preferred_element_type=jnp.float32)
    acc = acc + b_ref[0]
    if relu:
        acc = jnp.maximum(acc, 0.0)
    o_ref[...] = acc


def _cheb_matmul(t_stack, W, b, relu):
    nk, npad, cin = t_stack.shape
    cout = W.shape[2]
    grid = (npad // _BN,)
    body = functools.partial(_cheb_body, nk, relu)
    return pl.pallas_call(
        body,
        grid=grid,
        in_specs=[
            pl.BlockSpec((nk, _BN, cin), lambda i: (0, i, 0)),
            pl.BlockSpec((nk, cin, cout), lambda i: (0, 0, 0)),
            pl.BlockSpec((1, cout), lambda i: (0, 0)),
        ],
        out_specs=pl.BlockSpec((_BN, cout), lambda i: (i, 0)),
        out_shape=jax.ShapeDtypeStruct((npad, cout), jnp.float32),
    )(t_stack, W, b[None, :])


def kernel(x, edge_index, params1, params2, params3):
    n = x.shape[0] // 3
    npad = ((n + _BN - 1) // _BN) * _BN
    src = edge_index[0]
    dst = edge_index[1]

    mask = (src != dst).astype(jnp.float32)
    deg = jax.ops.segment_sum(mask, src, num_segments=n)
    deg_inv_sqrt = jnp.where(deg > 0, 1.0 / jnp.sqrt(jnp.maximum(deg, 1.0)),
                             0.0)
    norm = (-deg_inv_sqrt[src] * deg_inv_sqrt[dst] * mask)[:, None]

    def prop(t):
        return jax.ops.segment_sum(norm * t[src], dst, num_segments=n)

    nlayers = len(params1)
    h = jnp.concatenate([x[0::3], x[1::3], x[2::3]], axis=1)
    for i in range(nlayers):
        W1, b1 = params1[i]
        W2, b2 = params2[i]
        W3, b3 = params3[i]
        nk, ci, co = W1.shape
        Wbd = jnp.zeros((nk, 3 * ci, 3 * co), jnp.float32)
        Wbd = (Wbd.at[:, :ci, :co].set(W1)
                  .at[:, ci:2 * ci, co:2 * co].set(W2)
                  .at[:, 2 * ci:, 2 * co:].set(W3))
        bcat = jnp.concatenate([b1, b2, b3])
        tx = [h]
        for k in range(1, nk):
            if k == 1:
                tx.append(prop(h))
            else:
                tx.append(2.0 * prop(tx[-1]) - tx[-2])
        t_stack = jnp.stack(tx, axis=0)
        t_stack = jnp.pad(t_stack, ((0, 0), (0, npad - n), (0, 0)))
        out = _cheb_matmul(t_stack, Wbd, bcat, relu=(i < nlayers - 1))
        h = out[:n]
    return h.reshape(3 * n, 1)
